# baseline (speedup 1.0000x reference)
import functools

import jax
import jax.numpy as jnp
from jax import lax
from jax.experimental import pallas as pl
from jax.experimental.pallas import tpu as pltpu
from jax.experimental.pallas import tpu_sc as plsc

N = 10000
NP = 10240
E = 320000
CH = 128
NCORE = 2
NSUB = 16
NW = NCORE * NSUB
G = 80
NCHUNKP = NW * G
EPAD = NCHUNKP * CH - E
NBUF = 8
GDEP = 4
RPS = NP // NSUB
BR = 2000
GR = N // BR
BN_EPS = 1e-3

_mesh = plsc.VectorSubcoreMesh(
    core_axis_name="c", subcore_axis_name="s",
    num_cores=NCORE, num_subcores=NSUB)


def _make_sc_degree():
    @functools.partial(
        pl.kernel,
        out_type=jax.ShapeDtypeStruct((NCORE, NP, 16), jnp.float32),
        mesh=_mesh,
        scratch_types=[
            pltpu.VMEM((G, CH), jnp.int32),
            pltpu.VMEM((CH, 16), jnp.float32),
            pltpu.VMEM_SHARED((NP, 16), jnp.float32),
        ] + [pltpu.SemaphoreType.DMA] * NBUF,
        compiler_params=pltpu.CompilerParams(use_tc_tiling_on_sc=False),
        name="sc_degree",
    )
    def deg_kernel(dst_hbm, ones_hbm, zeros_hbm, out_hbm,
                   didx, ones_v, acc, *ssem):
        cid = lax.axis_index("c")
        sid = lax.axis_index("s")
        wid = sid * NCORE + cid
        base = sid * RPS
        pltpu.sync_copy(dst_hbm.at[pl.ds(wid * G, G)], didx)
        pltpu.sync_copy(ones_hbm, ones_v)
        pltpu.sync_copy(zeros_hbm, acc.at[pl.ds(base, RPS)])
        plsc.subcore_barrier()
        for b in range(NBUF):
            pltpu.async_copy(ones_v, acc.at[didx.at[b]], ssem[b], add=True)

        def group(gi, carry):
            for b in range(NBUF):
                t = gi * NBUF + b
                pltpu.make_async_copy(ones_v, acc.at[didx.at[t]],
                                      ssem[b]).wait()

                @pl.when(t + NBUF < G)
                def _():
                    pltpu.async_copy(ones_v, acc.at[didx.at[t + NBUF]],
                                     ssem[b], add=True)
            return carry

        lax.fori_loop(0, G // NBUF, group, 0)
        plsc.subcore_barrier()
        pltpu.sync_copy(acc.at[pl.ds(base, RPS)],
                        out_hbm.at[cid].at[pl.ds(base, RPS)])

    return deg_kernel


def _make_sc_scatter(d):
    @functools.partial(
        pl.kernel,
        out_type=jax.ShapeDtypeStruct((NCORE, NP, d), jnp.float32),
        mesh=_mesh,
        scratch_types=[
            pltpu.VMEM((G, CH), jnp.int32),
            pltpu.VMEM((G, CH), jnp.int32),
            pltpu.VMEM((NBUF, CH, d), jnp.float32),
            pltpu.VMEM_SHARED((NP, d), jnp.float32),
        ] + [pltpu.SemaphoreType.DMA] * (2 * NBUF),
        compiler_params=pltpu.CompilerParams(use_tc_tiling_on_sc=False),
        name=f"sc_scatter_{d}",
    )
    def sc_kernel(u_hbm, src_hbm, dst_hbm, zeros_hbm, out_hbm,
                  sidx, didx, rows, acc, *sems):
        gsem, ssem = sems[:NBUF], sems[NBUF:]
        cid = lax.axis_index("c")
        sid = lax.axis_index("s")
        wid = sid * NCORE + cid
        base = sid * RPS
        pltpu.sync_copy(src_hbm.at[pl.ds(wid * G, G)], sidx)
        pltpu.sync_copy(dst_hbm.at[pl.ds(wid * G, G)], didx)
        for b in range(GDEP):
            pltpu.async_copy(u_hbm.at[sidx.at[b]], rows.at[b], gsem[b])
        pltpu.sync_copy(zeros_hbm, acc.at[pl.ds(base, RPS)])
        plsc.subcore_barrier()

        def group(gi, carry):
            for b0 in range(NBUF):
                t = gi * NBUF + b0
                b = b0
                pltpu.make_async_copy(u_hbm.at[sidx.at[t]], rows.at[b],
                                      gsem[b]).wait()
                pltpu.async_copy(rows.at[b], acc.at[didx.at[t]], ssem[b],
                                 add=True)
                bn = (b0 + GDEP) % NBUF

                @pl.when(t >= GDEP)
                def _():
                    pltpu.make_async_copy(rows.at[bn], acc.at[didx.at[t]],
                                          ssem[bn]).wait()

                @pl.when(t + GDEP < G)
                def _():
                    pltpu.async_copy(u_hbm.at[sidx.at[t + GDEP]], rows.at[bn],
                                     gsem[bn])
            return carry

        lax.fori_loop(0, G // NBUF, group, 0)
        for k in range(GDEP):
            t = G - GDEP + k
            pltpu.make_async_copy(rows.at[t % NBUF], acc.at[didx.at[t]],
                                  ssem[t % NBUF]).wait()
        plsc.subcore_barrier()
        pltpu.sync_copy(acc.at[pl.ds(base, RPS)],
                        out_hbm.at[cid].at[pl.ds(base, RPS)])

    return sc_kernel


_SC_DEGREE = _make_sc_degree()
_SC_SCATTER = {d: _make_sc_scatter(d) for d in (16, 32, 64)}


def _mm_first(x, W, b, degp):
    dout = W.shape[1]

    def body(x_ref, w_ref, b_ref, deg_ref, u_ref, dis_ref):
        deg = deg_ref[0, :, 0:1] + deg_ref[1, :, 0:1] + 1.0
        dis = lax.rsqrt(deg)
        t = jnp.dot(x_ref[...], w_ref[...],
                    preferred_element_type=jnp.float32) + b_ref[...]
        u_ref[...] = t * dis
        dis_ref[...] = dis

    return pl.pallas_call(
        body,
        grid=(GR,),
        in_specs=[
            pl.BlockSpec((BR, 128), lambda i: (i, 0)),
            pl.BlockSpec((128, dout), lambda i: (0, 0)),
            pl.BlockSpec((1, dout), lambda i: (0, 0)),
            pl.BlockSpec((2, BR, 16), lambda i: (0, i, 0)),
        ],
        out_specs=[pl.BlockSpec((BR, dout), lambda i: (i, 0)),
                   pl.BlockSpec((BR, 1), lambda i: (i, 0))],
        out_shape=[jax.ShapeDtypeStruct((N, dout), jnp.float32),
                   jax.ShapeDtypeStruct((N, 1), jnp.float32)],
    )(x, W, b, degp)


def _combine_mm(S, u, dis, W, b):
    din = u.shape[1]
    dout = W.shape[1]

    def body(s_ref, u_ref, dis_ref, w_ref, b_ref, o_ref):
        h = dis_ref[...] * (s_ref[0] + s_ref[1] + u_ref[...])
        h = jnp.maximum(h, 0.0)
        t = jnp.dot(h, w_ref[...],
                    preferred_element_type=jnp.float32) + b_ref[...]
        o_ref[...] = t * dis_ref[...]

    return pl.pallas_call(
        body,
        grid=(GR,),
        in_specs=[
            pl.BlockSpec((2, BR, din), lambda i: (0, i, 0)),
            pl.BlockSpec((BR, din), lambda i: (i, 0)),
            pl.BlockSpec((BR, 1), lambda i: (i, 0)),
            pl.BlockSpec((din, dout), lambda i: (0, 0)),
            pl.BlockSpec((1, dout), lambda i: (0, 0)),
        ],
        out_specs=pl.BlockSpec((BR, dout), lambda i: (i, 0)),
        out_shape=jax.ShapeDtypeStruct((N, dout), jnp.float32),
    )(S, u, dis, W, b)


def _bn_mm(S, u, dis, g, be, W, b):
    din = u.shape[1]
    dout = W.shape[1]

    def body(s_ref, u_ref, dis_ref, g_ref, be_ref, w_ref, b_ref,
             o_ref, st_ref):
        p = pl.program_id(0)
        i = pl.program_id(1)
        h = dis_ref[...] * (s_ref[0] + s_ref[1] + u_ref[...])
        h = jnp.maximum(h, 0.0)

        @pl.when(p == 0)
        def _():
            part = jnp.concatenate([jnp.sum(h, 0, keepdims=True),
                                    jnp.sum(h * h, 0, keepdims=True)], axis=0)

            @pl.when(i == 0)
            def _():
                st_ref[...] = part

            @pl.when(i != 0)
            def _():
                st_ref[...] += part

        @pl.when(p == 1)
        def _():
            mean = st_ref[0:1] * (1.0 / N)
            var = st_ref[1:2] * (1.0 / N) - mean * mean
            inv = lax.rsqrt(var + BN_EPS)
            hb = (h - mean) * inv * g_ref[...] + be_ref[...]
            t = jnp.dot(hb, w_ref[...],
                        preferred_element_type=jnp.float32) + b_ref[...]
            o_ref[...] = t * dis_ref[...]

    return pl.pallas_call(
        body,
        grid=(2, GR),
        in_specs=[
            pl.BlockSpec((2, BR, din), lambda p, i: (0, i, 0)),
            pl.BlockSpec((BR, din), lambda p, i: (i, 0)),
            pl.BlockSpec((BR, 1), lambda p, i: (i, 0)),
            pl.BlockSpec((1, din), lambda p, i: (0, 0)),
            pl.BlockSpec((1, din), lambda p, i: (0, 0)),
            pl.BlockSpec((din, dout), lambda p, i: (0, 0)),
            pl.BlockSpec((1, dout), lambda p, i: (0, 0)),
        ],
        out_specs=pl.BlockSpec((BR, dout), lambda p, i: (i, 0)),
        out_shape=jax.ShapeDtypeStruct((N, dout), jnp.float32),
        scratch_shapes=[pltpu.VMEM((2, din), jnp.float32)],
    )(S, u, dis, g, be, W, b)


def _softmax_out(S, u, dis):
    def body(s_ref, u_ref, dis_ref, o_ref):
        lg = dis_ref[...] * (s_ref[0] + s_ref[1] + u_ref[...])
        col = lax.broadcasted_iota(jnp.int32, lg.shape, 1)
        lg = jnp.where(col < 10, lg, -jnp.inf)
        m = jnp.max(lg, axis=1, keepdims=True)
        e = jnp.exp(lg - m)
        p = e / jnp.sum(e, axis=1, keepdims=True)
        o_ref[...] = p[:, :10]

    return pl.pallas_call(
        body,
        grid=(GR,),
        in_specs=[
            pl.BlockSpec((2, BR, 16), lambda i: (0, i, 0)),
            pl.BlockSpec((BR, 16), lambda i: (i, 0)),
            pl.BlockSpec((BR, 1), lambda i: (i, 0)),
        ],
        out_specs=pl.BlockSpec((BR, 10), lambda i: (i, 0)),
        out_shape=jax.ShapeDtypeStruct((N, 10), jnp.float32),
    )(S, u, dis)


def kernel(x, edge_index, ids, W1, b1, W2, b2, g1, be1, W3, b3, W4, b4,
           g2, be2, W5, b5):
    pad = jnp.arange(EPAD, dtype=jnp.int32)
    src = jnp.concatenate([edge_index[0], (pad * 131) % N]).reshape(NCHUNKP, CH)
    dst = jnp.concatenate([edge_index[1], N + pad % (NP - N)]).reshape(NCHUNKP, CH)
    ones16 = jnp.ones((CH, 16), jnp.float32)
    z16 = jnp.zeros((RPS, 16), jnp.float32)
    z32 = jnp.zeros((RPS, 32), jnp.float32)
    z64 = jnp.zeros((RPS, 64), jnp.float32)

    degp = _SC_DEGREE(dst, ones16, z16)
    u1, dis = _mm_first(x, W1, b1.reshape(1, -1), degp)
    S1 = _SC_SCATTER[32](u1, src, dst, z32)
    u2 = _combine_mm(S1, u1, dis, W2, b2.reshape(1, -1))
    S2 = _SC_SCATTER[32](u2, src, dst, z32)
    u3 = _bn_mm(S2, u2, dis, g1.reshape(1, -1), be1.reshape(1, -1),
                W3, b3.reshape(1, -1))
    S3 = _SC_SCATTER[64](u3, src, dst, z64)
    u4 = _combine_mm(S3, u3, dis, W4, b4.reshape(1, -1))
    S4 = _SC_SCATTER[64](u4, src, dst, z64)
    W5p = jnp.pad(W5, ((0, 0), (0, 6)))
    b5p = jnp.pad(b5, (0, 6)).reshape(1, -1)
    u5 = _bn_mm(S4, u4, dis, g2.reshape(1, -1), be2.reshape(1, -1),
                W5p, b5p)
    S5 = _SC_SCATTER[16](u5, src, dst, z16)
    return _softmax_out(S5, u5, dis)


# speedup vs baseline: 1.0511x; 1.0511x over previous
import functools

import jax
import jax.numpy as jnp
from jax import lax
from jax.experimental import pallas as pl
from jax.experimental.pallas import tpu as pltpu
from jax.experimental.pallas import tpu_sc as plsc

N = 10000
NP = 10240
E = 320000
CH = 128
NCORE = 2
NSUB = 16
NW = NCORE * NSUB
G = 80
NCHUNKP = NW * G
EPAD = NCHUNKP * CH - E
GDEP = 8
RPS = NP // NSUB
BR = 2000
GR = N // BR
BN_EPS = 1e-3

_mesh = plsc.VectorSubcoreMesh(
    core_axis_name="c", subcore_axis_name="s",
    num_cores=NCORE, num_subcores=NSUB)


def _make_sc_degree():
    @functools.partial(
        pl.kernel,
        out_type=jax.ShapeDtypeStruct((NCORE, NP, 16), jnp.float32),
        mesh=_mesh,
        scratch_types=[
            pltpu.VMEM((G, CH), jnp.int32),
            pltpu.VMEM((CH, 16), jnp.float32),
            pltpu.VMEM_SHARED((NP, 16), jnp.float32),
        ] + [pltpu.SemaphoreType.DMA] * GDEP,
        compiler_params=pltpu.CompilerParams(use_tc_tiling_on_sc=False),
        name="sc_degree",
    )
    def deg_kernel(dst_hbm, ones_hbm, zeros_hbm, out_hbm,
                   didx, ones_v, acc, *ssem):
        cid = lax.axis_index("c")
        sid = lax.axis_index("s")
        wid = sid * NCORE + cid
        base = sid * RPS
        pltpu.sync_copy(dst_hbm.at[pl.ds(wid * G, G)], didx)
        pltpu.sync_copy(ones_hbm, ones_v)
        pltpu.sync_copy(zeros_hbm, acc.at[pl.ds(base, RPS)])
        plsc.subcore_barrier()
        for b in range(GDEP):
            pltpu.async_copy(ones_v, acc.at[didx.at[b]], ssem[b], add=True)

        def group(gi, carry):
            for b in range(GDEP):
                t = gi * GDEP + b
                pltpu.make_async_copy(ones_v, acc.at[didx.at[t]],
                                      ssem[b]).wait()

                @pl.when(t + GDEP < G)
                def _():
                    pltpu.async_copy(ones_v, acc.at[didx.at[t + GDEP]],
                                     ssem[b], add=True)
            return carry

        lax.fori_loop(0, G // GDEP, group, 0)
        plsc.subcore_barrier()
        pltpu.sync_copy(acc.at[pl.ds(base, RPS)],
                        out_hbm.at[cid].at[pl.ds(base, RPS)])

    return deg_kernel


def _make_sc_scatter(d):
    @functools.partial(
        pl.kernel,
        out_type=jax.ShapeDtypeStruct((NCORE, NP, d), jnp.float32),
        mesh=_mesh,
        scratch_types=[
            pltpu.VMEM((G, CH), jnp.int32),
            pltpu.VMEM((G, CH), jnp.int32),
            pltpu.VMEM((GDEP, CH, d), jnp.float32),
            pltpu.VMEM_SHARED((NP, d), jnp.float32),
        ] + [pltpu.SemaphoreType.DMA] * (2 * GDEP),
        compiler_params=pltpu.CompilerParams(use_tc_tiling_on_sc=False),
        name=f"sc_scatter_{d}",
    )
    def sc_kernel(u_hbm, src_hbm, dst_hbm, zeros_hbm, out_hbm,
                  sidx, didx, rows, acc, *sems):
        gsem, ssem = sems[:GDEP], sems[GDEP:]
        cid = lax.axis_index("c")
        sid = lax.axis_index("s")
        wid = sid * NCORE + cid
        base = sid * RPS
        pltpu.sync_copy(src_hbm.at[pl.ds(wid * G, G)], sidx)
        pltpu.sync_copy(dst_hbm.at[pl.ds(wid * G, G)], didx)
        for b in range(GDEP):
            pltpu.async_copy(u_hbm.at[sidx.at[b]], rows.at[b], gsem[b])
        pltpu.sync_copy(zeros_hbm, acc.at[pl.ds(base, RPS)])
        plsc.subcore_barrier()

        def group(gi, carry):
            for b in range(GDEP):
                t = gi * GDEP + b
                pltpu.make_async_copy(u_hbm.at[sidx.at[t]], rows.at[b],
                                      gsem[b]).wait()
                pltpu.async_copy(rows.at[b], acc.at[didx.at[t]], ssem[b],
                                 add=True)

                @pl.when(t + GDEP < G)
                def _():
                    pltpu.make_async_copy(rows.at[b], acc.at[didx.at[t]],
                                          ssem[b]).wait()
                    pltpu.async_copy(u_hbm.at[sidx.at[t + GDEP]], rows.at[b],
                                     gsem[b])
            return carry

        lax.fori_loop(0, G // GDEP, group, 0)
        for b in range(GDEP):
            pltpu.make_async_copy(rows.at[b], acc.at[didx.at[G - GDEP + b]],
                                  ssem[b]).wait()
        plsc.subcore_barrier()
        pltpu.sync_copy(acc.at[pl.ds(base, RPS)],
                        out_hbm.at[cid].at[pl.ds(base, RPS)])

    return sc_kernel


_SC_DEGREE = _make_sc_degree()
_SC_SCATTER = {d: _make_sc_scatter(d) for d in (16, 32, 64)}


def _mm_first(x, W, b, degp):
    dout = W.shape[1]

    def body(x_ref, w_ref, b_ref, deg_ref, u_ref, dis_ref):
        deg = deg_ref[0, :, 0:1] + deg_ref[1, :, 0:1] + 1.0
        dis = lax.rsqrt(deg)
        t = jnp.dot(x_ref[...], w_ref[...],
                    preferred_element_type=jnp.float32) + b_ref[...]
        u_ref[...] = t * dis
        dis_ref[...] = dis

    return pl.pallas_call(
        body,
        grid=(GR,),
        in_specs=[
            pl.BlockSpec((BR, 128), lambda i: (i, 0)),
            pl.BlockSpec((128, dout), lambda i: (0, 0)),
            pl.BlockSpec((1, dout), lambda i: (0, 0)),
            pl.BlockSpec((2, BR, 16), lambda i: (0, i, 0)),
        ],
        out_specs=[pl.BlockSpec((BR, dout), lambda i: (i, 0)),
                   pl.BlockSpec((BR, 1), lambda i: (i, 0))],
        out_shape=[jax.ShapeDtypeStruct((N, dout), jnp.float32),
                   jax.ShapeDtypeStruct((N, 1), jnp.float32)],
    )(x, W, b, degp)


def _combine_mm(S, u, dis, W, b):
    din = u.shape[1]
    dout = W.shape[1]

    def body(s_ref, u_ref, dis_ref, w_ref, b_ref, o_ref):
        h = dis_ref[...] * (s_ref[0] + s_ref[1] + u_ref[...])
        h = jnp.maximum(h, 0.0)
        t = jnp.dot(h, w_ref[...],
                    preferred_element_type=jnp.float32) + b_ref[...]
        o_ref[...] = t * dis_ref[...]

    return pl.pallas_call(
        body,
        grid=(GR,),
        in_specs=[
            pl.BlockSpec((2, BR, din), lambda i: (0, i, 0)),
            pl.BlockSpec((BR, din), lambda i: (i, 0)),
            pl.BlockSpec((BR, 1), lambda i: (i, 0)),
            pl.BlockSpec((din, dout), lambda i: (0, 0)),
            pl.BlockSpec((1, dout), lambda i: (0, 0)),
        ],
        out_specs=pl.BlockSpec((BR, dout), lambda i: (i, 0)),
        out_shape=jax.ShapeDtypeStruct((N, dout), jnp.float32),
    )(S, u, dis, W, b)


def _bn_mm(S, u, dis, g, be, W, b):
    din = u.shape[1]
    dout = W.shape[1]

    def body(s_ref, u_ref, dis_ref, g_ref, be_ref, w_ref, b_ref,
             o_ref, st_ref):
        p = pl.program_id(0)
        i = pl.program_id(1)
        h = dis_ref[...] * (s_ref[0] + s_ref[1] + u_ref[...])
        h = jnp.maximum(h, 0.0)

        @pl.when(p == 0)
        def _():
            part = jnp.concatenate([jnp.sum(h, 0, keepdims=True),
                                    jnp.sum(h * h, 0, keepdims=True)], axis=0)

            @pl.when(i == 0)
            def _():
                st_ref[...] = part

            @pl.when(i != 0)
            def _():
                st_ref[...] += part

        @pl.when(p == 1)
        def _():
            mean = st_ref[0:1] * (1.0 / N)
            var = st_ref[1:2] * (1.0 / N) - mean * mean
            inv = lax.rsqrt(var + BN_EPS)
            hb = (h - mean) * inv * g_ref[...] + be_ref[...]
            t = jnp.dot(hb, w_ref[...],
                        preferred_element_type=jnp.float32) + b_ref[...]
            o_ref[...] = t * dis_ref[...]

    return pl.pallas_call(
        body,
        grid=(2, GR),
        in_specs=[
            pl.BlockSpec((2, BR, din), lambda p, i: (0, i, 0)),
            pl.BlockSpec((BR, din), lambda p, i: (i, 0)),
            pl.BlockSpec((BR, 1), lambda p, i: (i, 0)),
            pl.BlockSpec((1, din), lambda p, i: (0, 0)),
            pl.BlockSpec((1, din), lambda p, i: (0, 0)),
            pl.BlockSpec((din, dout), lambda p, i: (0, 0)),
            pl.BlockSpec((1, dout), lambda p, i: (0, 0)),
        ],
        out_specs=pl.BlockSpec((BR, dout), lambda p, i: (i, 0)),
        out_shape=jax.ShapeDtypeStruct((N, dout), jnp.float32),
        scratch_shapes=[pltpu.VMEM((2, din), jnp.float32)],
    )(S, u, dis, g, be, W, b)


def _softmax_out(S, u, dis):
    def body(s_ref, u_ref, dis_ref, o_ref):
        lg = dis_ref[...] * (s_ref[0] + s_ref[1] + u_ref[...])
        col = lax.broadcasted_iota(jnp.int32, lg.shape, 1)
        lg = jnp.where(col < 10, lg, -jnp.inf)
        m = jnp.max(lg, axis=1, keepdims=True)
        e = jnp.exp(lg - m)
        p = e / jnp.sum(e, axis=1, keepdims=True)
        o_ref[...] = p[:, :10]

    return pl.pallas_call(
        body,
        grid=(GR,),
        in_specs=[
            pl.BlockSpec((2, BR, 16), lambda i: (0, i, 0)),
            pl.BlockSpec((BR, 16), lambda i: (i, 0)),
            pl.BlockSpec((BR, 1), lambda i: (i, 0)),
        ],
        out_specs=pl.BlockSpec((BR, 10), lambda i: (i, 0)),
        out_shape=jax.ShapeDtypeStruct((N, 10), jnp.float32),
    )(S, u, dis)


def kernel(x, edge_index, ids, W1, b1, W2, b2, g1, be1, W3, b3, W4, b4,
           g2, be2, W5, b5):
    pad = jnp.arange(EPAD, dtype=jnp.int32)
    src = jnp.concatenate([edge_index[0], (pad * 131) % N]).reshape(NCHUNKP, CH)
    dst = jnp.concatenate([edge_index[1], N + pad % (NP - N)]).reshape(NCHUNKP, CH)
    ones16 = jnp.ones((CH, 16), jnp.float32)
    z16 = jnp.zeros((RPS, 16), jnp.float32)
    z32 = jnp.zeros((RPS, 32), jnp.float32)
    z64 = jnp.zeros((RPS, 64), jnp.float32)

    degp = _SC_DEGREE(dst, ones16, z16)
    u1, dis = _mm_first(x, W1, b1.reshape(1, -1), degp)
    S1 = _SC_SCATTER[32](u1, src, dst, z32)
    u2 = _combine_mm(S1, u1, dis, W2, b2.reshape(1, -1))
    S2 = _SC_SCATTER[32](u2, src, dst, z32)
    u3 = _bn_mm(S2, u2, dis, g1.reshape(1, -1), be1.reshape(1, -1),
                W3, b3.reshape(1, -1))
    S3 = _SC_SCATTER[64](u3, src, dst, z64)
    u4 = _combine_mm(S3, u3, dis, W4, b4.reshape(1, -1))
    S4 = _SC_SCATTER[64](u4, src, dst, z64)
    W5p = jnp.pad(W5, ((0, 0), (0, 6)))
    b5p = jnp.pad(b5, (0, 6)).reshape(1, -1)
    u5 = _bn_mm(S4, u4, dis, g2.reshape(1, -1), be2.reshape(1, -1),
                W5p, b5p)
    S5 = _SC_SCATTER[16](u5, src, dst, z16)
    return _softmax_out(S5, u5, dis)


# speedup vs baseline: 1.0667x; 1.0149x over previous
import functools

import jax
import jax.numpy as jnp
from jax import lax
from jax.experimental import pallas as pl
from jax.experimental.pallas import tpu as pltpu
from jax.experimental.pallas import tpu_sc as plsc

N = 10000
NP = 10240
E = 320000
CH = 256
NCORE = 2
NSUB = 16
NW = NCORE * NSUB
G = 40
NCHUNKP = NW * G
EPAD = NCHUNKP * CH - E
GDEP = 4
RPS = NP // NSUB
BR = 10000
GR = N // BR
BN_EPS = 1e-3

_mesh = plsc.VectorSubcoreMesh(
    core_axis_name="c", subcore_axis_name="s",
    num_cores=NCORE, num_subcores=NSUB)


def _make_sc_degree():
    @functools.partial(
        pl.kernel,
        out_type=jax.ShapeDtypeStruct((NCORE, NP, 16), jnp.float32),
        mesh=_mesh,
        scratch_types=[
            pltpu.VMEM((G, CH), jnp.int32),
            pltpu.VMEM((CH, 16), jnp.float32),
            pltpu.VMEM_SHARED((NP, 16), jnp.float32),
        ] + [pltpu.SemaphoreType.DMA] * GDEP,
        compiler_params=pltpu.CompilerParams(use_tc_tiling_on_sc=False),
        name="sc_degree",
    )
    def deg_kernel(dst_hbm, ones_hbm, zeros_hbm, out_hbm,
                   didx, ones_v, acc, *ssem):
        cid = lax.axis_index("c")
        sid = lax.axis_index("s")
        wid = sid * NCORE + cid
        base = sid * RPS
        pltpu.sync_copy(dst_hbm.at[pl.ds(wid * G, G)], didx)
        pltpu.sync_copy(ones_hbm, ones_v)
        pltpu.sync_copy(zeros_hbm, acc.at[pl.ds(base, RPS)])
        plsc.subcore_barrier()
        for b in range(GDEP):
            pltpu.async_copy(ones_v, acc.at[didx.at[b]], ssem[b], add=True)

        def group(gi, carry):
            for b in range(GDEP):
                t = gi * GDEP + b
                pltpu.make_async_copy(ones_v, acc.at[didx.at[t]],
                                      ssem[b]).wait()

                @pl.when(t + GDEP < G)
                def _():
                    pltpu.async_copy(ones_v, acc.at[didx.at[t + GDEP]],
                                     ssem[b], add=True)
            return carry

        lax.fori_loop(0, G // GDEP, group, 0)
        plsc.subcore_barrier()
        pltpu.sync_copy(acc.at[pl.ds(base, RPS)],
                        out_hbm.at[cid].at[pl.ds(base, RPS)])

    return deg_kernel


def _make_sc_scatter(d):
    @functools.partial(
        pl.kernel,
        out_type=jax.ShapeDtypeStruct((NCORE, NP, d), jnp.float32),
        mesh=_mesh,
        scratch_types=[
            pltpu.VMEM((G, CH), jnp.int32),
            pltpu.VMEM((G, CH), jnp.int32),
            pltpu.VMEM((GDEP, CH, d), jnp.float32),
            pltpu.VMEM_SHARED((NP, d), jnp.float32),
        ] + [pltpu.SemaphoreType.DMA] * (2 * GDEP),
        compiler_params=pltpu.CompilerParams(use_tc_tiling_on_sc=False),
        name=f"sc_scatter_{d}",
    )
    def sc_kernel(u_hbm, src_hbm, dst_hbm, zeros_hbm, out_hbm,
                  sidx, didx, rows, acc, *sems):
        gsem, ssem = sems[:GDEP], sems[GDEP:]
        cid = lax.axis_index("c")
        sid = lax.axis_index("s")
        wid = sid * NCORE + cid
        base = sid * RPS
        pltpu.sync_copy(src_hbm.at[pl.ds(wid * G, G)], sidx)
        pltpu.sync_copy(dst_hbm.at[pl.ds(wid * G, G)], didx)
        for b in range(GDEP):
            pltpu.async_copy(u_hbm.at[sidx.at[b]], rows.at[b], gsem[b])
        pltpu.sync_copy(zeros_hbm, acc.at[pl.ds(base, RPS)])
        plsc.subcore_barrier()

        def group(gi, carry):
            for b in range(GDEP):
                t = gi * GDEP + b
                pltpu.make_async_copy(u_hbm.at[sidx.at[t]], rows.at[b],
                                      gsem[b]).wait()
                pltpu.async_copy(rows.at[b], acc.at[didx.at[t]], ssem[b],
                                 add=True)

                @pl.when(t + GDEP < G)
                def _():
                    pltpu.make_async_copy(rows.at[b], acc.at[didx.at[t]],
                                          ssem[b]).wait()
                    pltpu.async_copy(u_hbm.at[sidx.at[t + GDEP]], rows.at[b],
                                     gsem[b])
            return carry

        lax.fori_loop(0, G // GDEP, group, 0)
        for b in range(GDEP):
            pltpu.make_async_copy(rows.at[b], acc.at[didx.at[G - GDEP + b]],
                                  ssem[b]).wait()
        plsc.subcore_barrier()
        pltpu.sync_copy(acc.at[pl.ds(base, RPS)],
                        out_hbm.at[cid].at[pl.ds(base, RPS)])

    return sc_kernel


_SC_DEGREE = _make_sc_degree()
_SC_SCATTER = {d: _make_sc_scatter(d) for d in (16, 32, 64)}


def _mm_first(x, W, b, degp):
    dout = W.shape[1]

    def body(x_ref, w_ref, b_ref, deg_ref, u_ref, dis_ref):
        deg = deg_ref[0, :, 0:1] + deg_ref[1, :, 0:1] + 1.0
        dis = lax.rsqrt(deg)
        t = jnp.dot(x_ref[...], w_ref[...],
                    preferred_element_type=jnp.float32) + b_ref[...]
        u_ref[...] = t * dis
        dis_ref[...] = dis

    return pl.pallas_call(
        body,
        grid=(GR,),
        in_specs=[
            pl.BlockSpec((BR, 128), lambda i: (i, 0)),
            pl.BlockSpec((128, dout), lambda i: (0, 0)),
            pl.BlockSpec((1, dout), lambda i: (0, 0)),
            pl.BlockSpec((2, BR, 16), lambda i: (0, i, 0)),
        ],
        out_specs=[pl.BlockSpec((BR, dout), lambda i: (i, 0)),
                   pl.BlockSpec((BR, 1), lambda i: (i, 0))],
        out_shape=[jax.ShapeDtypeStruct((N, dout), jnp.float32),
                   jax.ShapeDtypeStruct((N, 1), jnp.float32)],
    )(x, W, b, degp)


def _combine_mm(S, u, dis, W, b):
    din = u.shape[1]
    dout = W.shape[1]

    def body(s_ref, u_ref, dis_ref, w_ref, b_ref, o_ref):
        h = dis_ref[...] * (s_ref[0] + s_ref[1] + u_ref[...])
        h = jnp.maximum(h, 0.0)
        t = jnp.dot(h, w_ref[...],
                    preferred_element_type=jnp.float32) + b_ref[...]
        o_ref[...] = t * dis_ref[...]

    return pl.pallas_call(
        body,
        grid=(GR,),
        in_specs=[
            pl.BlockSpec((2, BR, din), lambda i: (0, i, 0)),
            pl.BlockSpec((BR, din), lambda i: (i, 0)),
            pl.BlockSpec((BR, 1), lambda i: (i, 0)),
            pl.BlockSpec((din, dout), lambda i: (0, 0)),
            pl.BlockSpec((1, dout), lambda i: (0, 0)),
        ],
        out_specs=pl.BlockSpec((BR, dout), lambda i: (i, 0)),
        out_shape=jax.ShapeDtypeStruct((N, dout), jnp.float32),
    )(S, u, dis, W, b)


def _bn_mm(S, u, dis, g, be, W, b):
    din = u.shape[1]
    dout = W.shape[1]

    def body(s_ref, u_ref, dis_ref, g_ref, be_ref, w_ref, b_ref,
             o_ref, st_ref):
        p = pl.program_id(0)
        i = pl.program_id(1)
        h = dis_ref[...] * (s_ref[0] + s_ref[1] + u_ref[...])
        h = jnp.maximum(h, 0.0)

        @pl.when(p == 0)
        def _():
            part = jnp.concatenate([jnp.sum(h, 0, keepdims=True),
                                    jnp.sum(h * h, 0, keepdims=True)], axis=0)

            @pl.when(i == 0)
            def _():
                st_ref[...] = part

            @pl.when(i != 0)
            def _():
                st_ref[...] += part

        @pl.when(p == 1)
        def _():
            mean = st_ref[0:1] * (1.0 / N)
            var = st_ref[1:2] * (1.0 / N) - mean * mean
            inv = lax.rsqrt(var + BN_EPS)
            hb = (h - mean) * inv * g_ref[...] + be_ref[...]
            t = jnp.dot(hb, w_ref[...],
                        preferred_element_type=jnp.float32) + b_ref[...]
            o_ref[...] = t * dis_ref[...]

    return pl.pallas_call(
        body,
        grid=(2, GR),
        in_specs=[
            pl.BlockSpec((2, BR, din), lambda p, i: (0, i, 0)),
            pl.BlockSpec((BR, din), lambda p, i: (i, 0)),
            pl.BlockSpec((BR, 1), lambda p, i: (i, 0)),
            pl.BlockSpec((1, din), lambda p, i: (0, 0)),
            pl.BlockSpec((1, din), lambda p, i: (0, 0)),
            pl.BlockSpec((din, dout), lambda p, i: (0, 0)),
            pl.BlockSpec((1, dout), lambda p, i: (0, 0)),
        ],
        out_specs=pl.BlockSpec((BR, dout), lambda p, i: (i, 0)),
        out_shape=jax.ShapeDtypeStruct((N, dout), jnp.float32),
        scratch_shapes=[pltpu.VMEM((2, din), jnp.float32)],
    )(S, u, dis, g, be, W, b)


def _softmax_out(S, u, dis):
    def body(s_ref, u_ref, dis_ref, o_ref):
        lg = dis_ref[...] * (s_ref[0] + s_ref[1] + u_ref[...])
        col = lax.broadcasted_iota(jnp.int32, lg.shape, 1)
        lg = jnp.where(col < 10, lg, -jnp.inf)
        m = jnp.max(lg, axis=1, keepdims=True)
        e = jnp.exp(lg - m)
        p = e / jnp.sum(e, axis=1, keepdims=True)
        o_ref[...] = p[:, :10]

    return pl.pallas_call(
        body,
        grid=(GR,),
        in_specs=[
            pl.BlockSpec((2, BR, 16), lambda i: (0, i, 0)),
            pl.BlockSpec((BR, 16), lambda i: (i, 0)),
            pl.BlockSpec((BR, 1), lambda i: (i, 0)),
        ],
        out_specs=pl.BlockSpec((BR, 10), lambda i: (i, 0)),
        out_shape=jax.ShapeDtypeStruct((N, 10), jnp.float32),
    )(S, u, dis)


def kernel(x, edge_index, ids, W1, b1, W2, b2, g1, be1, W3, b3, W4, b4,
           g2, be2, W5, b5):
    pad = jnp.arange(EPAD, dtype=jnp.int32)
    src = jnp.concatenate([edge_index[0], (pad * 131) % N]).reshape(NCHUNKP, CH)
    dst = jnp.concatenate([edge_index[1], N + pad % (NP - N)]).reshape(NCHUNKP, CH)
    ones16 = jnp.ones((CH, 16), jnp.float32)
    z16 = jnp.zeros((RPS, 16), jnp.float32)
    z32 = jnp.zeros((RPS, 32), jnp.float32)
    z64 = jnp.zeros((RPS, 64), jnp.float32)

    degp = _SC_DEGREE(dst, ones16, z16)
    u1, dis = _mm_first(x, W1, b1.reshape(1, -1), degp)
    S1 = _SC_SCATTER[32](u1, src, dst, z32)
    u2 = _combine_mm(S1, u1, dis, W2, b2.reshape(1, -1))
    S2 = _SC_SCATTER[32](u2, src, dst, z32)
    u3 = _bn_mm(S2, u2, dis, g1.reshape(1, -1), be1.reshape(1, -1),
                W3, b3.reshape(1, -1))
    S3 = _SC_SCATTER[64](u3, src, dst, z64)
    u4 = _combine_mm(S3, u3, dis, W4, b4.reshape(1, -1))
    S4 = _SC_SCATTER[64](u4, src, dst, z64)
    W5p = jnp.pad(W5, ((0, 0), (0, 6)))
    b5p = jnp.pad(b5, (0, 6)).reshape(1, -1)
    u5 = _bn_mm(S4, u4, dis, g2.reshape(1, -1), be2.reshape(1, -1),
                W5p, b5p)
    S5 = _SC_SCATTER[16](u5, src, dst, z16)
    return _softmax_out(S5, u5, dis)


# speedup vs baseline: 1.0769x; 1.0095x over previous
import functools

import jax
import jax.numpy as jnp
from jax import lax
from jax.experimental import pallas as pl
from jax.experimental.pallas import tpu as pltpu
from jax.experimental.pallas import tpu_sc as plsc

N = 10000
NP = 10240
E = 320000
CH = 128
NCORE = 2
NSUB = 16
NW = NCORE * NSUB
G = 80
NCHUNKP = NW * G
NCHUNK = E // CH
GREM = NCHUNK - (NW - 1) * G
EPAD = (G - GREM) * CH
GDEP = 8
RPS = NP // NSUB
BR = 10000
GR = N // BR
BN_EPS = 1e-3

_mesh = plsc.VectorSubcoreMesh(
    core_axis_name="c", subcore_axis_name="s",
    num_cores=NCORE, num_subcores=NSUB)


def _make_sc_degree():
    @functools.partial(
        pl.kernel,
        out_type=jax.ShapeDtypeStruct((NCORE, NP, 16), jnp.float32),
        mesh=_mesh,
        scratch_types=[
            pltpu.VMEM((G, CH), jnp.int32),
            pltpu.VMEM((CH, 16), jnp.float32),
            pltpu.VMEM_SHARED((NP, 16), jnp.float32),
        ] + [pltpu.SemaphoreType.DMA] * GDEP,
        compiler_params=pltpu.CompilerParams(use_tc_tiling_on_sc=False),
        name="sc_degree",
    )
    def deg_kernel(dst_hbm, dpad_hbm, ones_hbm, zeros_hbm, out_hbm,
                   didx, ones_v, acc, *ssem):
        cid = lax.axis_index("c")
        sid = lax.axis_index("s")
        wid = sid * NCORE + cid
        base = sid * RPS

        @pl.when(wid < NW - 1)
        def _():
            pltpu.sync_copy(dst_hbm.at[pl.ds(wid * G, G)], didx)

        @pl.when(wid == NW - 1)
        def _():
            pltpu.sync_copy(dst_hbm.at[pl.ds((NW - 1) * G, GREM)],
                            didx.at[pl.ds(0, GREM)])
            pltpu.sync_copy(dpad_hbm, didx.at[pl.ds(GREM, G - GREM)])
        pltpu.sync_copy(ones_hbm, ones_v)
        pltpu.sync_copy(zeros_hbm, acc.at[pl.ds(base, RPS)])
        plsc.subcore_barrier()
        for b in range(GDEP):
            pltpu.async_copy(ones_v, acc.at[didx.at[b]], ssem[b], add=True)

        def group(gi, carry):
            for b in range(GDEP):
                t = gi * GDEP + b
                pltpu.make_async_copy(ones_v, acc.at[didx.at[t]],
                                      ssem[b]).wait()

                @pl.when(t + GDEP < G)
                def _():
                    pltpu.async_copy(ones_v, acc.at[didx.at[t + GDEP]],
                                     ssem[b], add=True)
            return carry

        lax.fori_loop(0, G // GDEP, group, 0)
        plsc.subcore_barrier()
        pltpu.sync_copy(acc.at[pl.ds(base, RPS)],
                        out_hbm.at[cid].at[pl.ds(base, RPS)])

    return deg_kernel


def _make_sc_scatter(d):
    @functools.partial(
        pl.kernel,
        out_type=jax.ShapeDtypeStruct((NCORE, NP, d), jnp.float32),
        mesh=_mesh,
        scratch_types=[
            pltpu.VMEM((G, CH), jnp.int32),
            pltpu.VMEM((G, CH), jnp.int32),
            pltpu.VMEM((GDEP, CH, d), jnp.float32),
            pltpu.VMEM_SHARED((NP, d), jnp.float32),
        ] + [pltpu.SemaphoreType.DMA] * (2 * GDEP),
        compiler_params=pltpu.CompilerParams(use_tc_tiling_on_sc=False),
        name=f"sc_scatter_{d}",
    )
    def sc_kernel(u_hbm, src_hbm, dst_hbm, spad_hbm, dpad_hbm, zeros_hbm,
                  out_hbm, sidx, didx, rows, acc, *sems):
        gsem, ssem = sems[:GDEP], sems[GDEP:]
        cid = lax.axis_index("c")
        sid = lax.axis_index("s")
        wid = sid * NCORE + cid
        base = sid * RPS

        @pl.when(wid < NW - 1)
        def _():
            pltpu.sync_copy(src_hbm.at[pl.ds(wid * G, G)], sidx)
            pltpu.sync_copy(dst_hbm.at[pl.ds(wid * G, G)], didx)

        @pl.when(wid == NW - 1)
        def _():
            pltpu.sync_copy(src_hbm.at[pl.ds((NW - 1) * G, GREM)],
                            sidx.at[pl.ds(0, GREM)])
            pltpu.sync_copy(dst_hbm.at[pl.ds((NW - 1) * G, GREM)],
                            didx.at[pl.ds(0, GREM)])
            pltpu.sync_copy(spad_hbm, sidx.at[pl.ds(GREM, G - GREM)])
            pltpu.sync_copy(dpad_hbm, didx.at[pl.ds(GREM, G - GREM)])
        for b in range(GDEP):
            pltpu.async_copy(u_hbm.at[sidx.at[b]], rows.at[b], gsem[b])
        pltpu.sync_copy(zeros_hbm, acc.at[pl.ds(base, RPS)])
        plsc.subcore_barrier()

        def group(gi, carry):
            for b in range(GDEP):
                t = gi * GDEP + b
                pltpu.make_async_copy(u_hbm.at[sidx.at[t]], rows.at[b],
                                      gsem[b]).wait()
                pltpu.async_copy(rows.at[b], acc.at[didx.at[t]], ssem[b],
                                 add=True)

                @pl.when(t + GDEP < G)
                def _():
                    pltpu.make_async_copy(rows.at[b], acc.at[didx.at[t]],
                                          ssem[b]).wait()
                    pltpu.async_copy(u_hbm.at[sidx.at[t + GDEP]], rows.at[b],
                                     gsem[b])
            return carry

        lax.fori_loop(0, G // GDEP, group, 0)
        for b in range(GDEP):
            pltpu.make_async_copy(rows.at[b], acc.at[didx.at[G - GDEP + b]],
                                  ssem[b]).wait()
        plsc.subcore_barrier()
        pltpu.sync_copy(acc.at[pl.ds(base, RPS)],
                        out_hbm.at[cid].at[pl.ds(base, RPS)])

    return sc_kernel


_SC_DEGREE = _make_sc_degree()
_SC_SCATTER = {d: _make_sc_scatter(d) for d in (16, 32, 64)}


def _mm_first(x, W, b, degp):
    dout = W.shape[1]

    def body(x_ref, w_ref, b_ref, deg_ref, u_ref, dis_ref):
        deg = deg_ref[0, :, 0:1] + deg_ref[1, :, 0:1] + 1.0
        dis = lax.rsqrt(deg)
        t = jnp.dot(x_ref[...], w_ref[...],
                    preferred_element_type=jnp.float32) + b_ref[...]
        u_ref[...] = t * dis
        dis_ref[...] = dis

    return pl.pallas_call(
        body,
        grid=(GR,),
        in_specs=[
            pl.BlockSpec((BR, 128), lambda i: (i, 0)),
            pl.BlockSpec((128, dout), lambda i: (0, 0)),
            pl.BlockSpec((1, dout), lambda i: (0, 0)),
            pl.BlockSpec((2, BR, 16), lambda i: (0, i, 0)),
        ],
        out_specs=[pl.BlockSpec((BR, dout), lambda i: (i, 0)),
                   pl.BlockSpec((BR, 1), lambda i: (i, 0))],
        out_shape=[jax.ShapeDtypeStruct((N, dout), jnp.float32),
                   jax.ShapeDtypeStruct((N, 1), jnp.float32)],
    )(x, W, b, degp)


def _combine_mm(S, u, dis, W, b):
    din = u.shape[1]
    dout = W.shape[1]

    def body(s_ref, u_ref, dis_ref, w_ref, b_ref, o_ref):
        h = dis_ref[...] * (s_ref[0] + s_ref[1] + u_ref[...])
        h = jnp.maximum(h, 0.0)
        t = jnp.dot(h, w_ref[...],
                    preferred_element_type=jnp.float32) + b_ref[...]
        o_ref[...] = t * dis_ref[...]

    return pl.pallas_call(
        body,
        grid=(GR,),
        in_specs=[
            pl.BlockSpec((2, BR, din), lambda i: (0, i, 0)),
            pl.BlockSpec((BR, din), lambda i: (i, 0)),
            pl.BlockSpec((BR, 1), lambda i: (i, 0)),
            pl.BlockSpec((din, dout), lambda i: (0, 0)),
            pl.BlockSpec((1, dout), lambda i: (0, 0)),
        ],
        out_specs=pl.BlockSpec((BR, dout), lambda i: (i, 0)),
        out_shape=jax.ShapeDtypeStruct((N, dout), jnp.float32),
    )(S, u, dis, W, b)


def _bn_mm(S, u, dis, g, be, W, b):
    din = u.shape[1]
    dout = W.shape[1]

    def body(s_ref, u_ref, dis_ref, g_ref, be_ref, w_ref, b_ref,
             o_ref, st_ref):
        p = pl.program_id(0)
        i = pl.program_id(1)
        h = dis_ref[...] * (s_ref[0] + s_ref[1] + u_ref[...])
        h = jnp.maximum(h, 0.0)

        @pl.when(p == 0)
        def _():
            part = jnp.concatenate([jnp.sum(h, 0, keepdims=True),
                                    jnp.sum(h * h, 0, keepdims=True)], axis=0)

            @pl.when(i == 0)
            def _():
                st_ref[...] = part

            @pl.when(i != 0)
            def _():
                st_ref[...] += part

        @pl.when(p == 1)
        def _():
            mean = st_ref[0:1] * (1.0 / N)
            var = st_ref[1:2] * (1.0 / N) - mean * mean
            inv = lax.rsqrt(var + BN_EPS)
            hb = (h - mean) * inv * g_ref[...] + be_ref[...]
            t = jnp.dot(hb, w_ref[...],
                        preferred_element_type=jnp.float32) + b_ref[...]
            o_ref[...] = t * dis_ref[...]

    return pl.pallas_call(
        body,
        grid=(2, GR),
        in_specs=[
            pl.BlockSpec((2, BR, din), lambda p, i: (0, i, 0)),
            pl.BlockSpec((BR, din), lambda p, i: (i, 0)),
            pl.BlockSpec((BR, 1), lambda p, i: (i, 0)),
            pl.BlockSpec((1, din), lambda p, i: (0, 0)),
            pl.BlockSpec((1, din), lambda p, i: (0, 0)),
            pl.BlockSpec((din, dout), lambda p, i: (0, 0)),
            pl.BlockSpec((1, dout), lambda p, i: (0, 0)),
        ],
        out_specs=pl.BlockSpec((BR, dout), lambda p, i: (i, 0)),
        out_shape=jax.ShapeDtypeStruct((N, dout), jnp.float32),
        scratch_shapes=[pltpu.VMEM((2, din), jnp.float32)],
    )(S, u, dis, g, be, W, b)


def _softmax_out(S, u, dis):
    def body(s_ref, u_ref, dis_ref, o_ref):
        lg = dis_ref[...] * (s_ref[0] + s_ref[1] + u_ref[...])
        col = lax.broadcasted_iota(jnp.int32, lg.shape, 1)
        lg = jnp.where(col < 10, lg, -jnp.inf)
        m = jnp.max(lg, axis=1, keepdims=True)
        e = jnp.exp(lg - m)
        p = e / jnp.sum(e, axis=1, keepdims=True)
        o_ref[...] = p[:, :10]

    return pl.pallas_call(
        body,
        grid=(GR,),
        in_specs=[
            pl.BlockSpec((2, BR, 16), lambda i: (0, i, 0)),
            pl.BlockSpec((BR, 16), lambda i: (i, 0)),
            pl.BlockSpec((BR, 1), lambda i: (i, 0)),
        ],
        out_specs=pl.BlockSpec((BR, 10), lambda i: (i, 0)),
        out_shape=jax.ShapeDtypeStruct((N, 10), jnp.float32),
    )(S, u, dis)


def kernel(x, edge_index, ids, W1, b1, W2, b2, g1, be1, W3, b3, W4, b4,
           g2, be2, W5, b5):
    src = edge_index[0].reshape(NCHUNK, CH)
    dst = edge_index[1].reshape(NCHUNK, CH)
    pad = jnp.arange(EPAD, dtype=jnp.int32)
    spad = ((pad * 131) % N).reshape(G - GREM, CH)
    dpad = (N + pad % (NP - N)).reshape(G - GREM, CH)
    ones16 = jnp.ones((CH, 16), jnp.float32)
    z16 = jnp.zeros((RPS, 16), jnp.float32)
    z32 = jnp.zeros((RPS, 32), jnp.float32)
    z64 = jnp.zeros((RPS, 64), jnp.float32)

    degp = _SC_DEGREE(dst, dpad, ones16, z16)
    u1, dis = _mm_first(x, W1, b1.reshape(1, -1), degp)
    S1 = _SC_SCATTER[32](u1, src, dst, spad, dpad, z32)
    u2 = _combine_mm(S1, u1, dis, W2, b2.reshape(1, -1))
    S2 = _SC_SCATTER[32](u2, src, dst, spad, dpad, z32)
    u3 = _bn_mm(S2, u2, dis, g1.reshape(1, -1), be1.reshape(1, -1),
                W3, b3.reshape(1, -1))
    S3 = _SC_SCATTER[64](u3, src, dst, spad, dpad, z64)
    u4 = _combine_mm(S3, u3, dis, W4, b4.reshape(1, -1))
    S4 = _SC_SCATTER[64](u4, src, dst, spad, dpad, z64)
    W5p = jnp.pad(W5, ((0, 0), (0, 6)))
    b5p = jnp.pad(b5, (0, 6)).reshape(1, -1)
    u5 = _bn_mm(S4, u4, dis, g2.reshape(1, -1), be2.reshape(1, -1),
                W5p, b5p)
    S5 = _SC_SCATTER[16](u5, src, dst, spad, dpad, z16)
    return _softmax_out(S5, u5, dis)
